# baseline (speedup 1.0000x reference)
import functools

import jax
import jax.numpy as jnp
from jax import lax
from jax.experimental import pallas as pl
from jax.experimental.pallas import tpu as pltpu
from jax.experimental.pallas import tpu_sc as plsc

_L = 16
_W = 128


def _nn_gather_body(n_grid, n_batch, b_per_w,
                    inp_hbm, ts_hbm, ys_hbm, out_hbm,
                    inp_v, idx_v, g_v, y_v, sem_in, sem_g):
    wid = lax.axis_index("s")
    base = wid * b_per_w
    pltpu.async_copy(inp_hbm.at[pl.ds(base, b_per_w)], inp_v, sem_in).wait()

    j_max = n_grid - 3
    inv_dt = jnp.float32((n_grid - 1) / 100.0)
    dt = jnp.float32(100.0 / (n_grid - 1))
    lane = lax.iota(jnp.int32, _L)

    gathers = []
    for c in range(b_per_w // _L):
        x = inp_v[pl.ds(c * _L, _L)]
        f = (x * inv_dt).astype(jnp.int32) - 1
        m0 = jnp.clip(f, 0, j_max)
        m1 = jnp.clip(f + 1, 0, j_max)
        d0 = jnp.abs(x - (m0 + 1).astype(jnp.float32) * dt)
        d1 = jnp.abs(x - (m1 + 1).astype(jnp.float32) * dt)
        idx_v[pl.ds(c * _L, _L)] = jnp.where(d1 < d0, m1, m0) + 1
        if (c + 1) % (_W // _L) == 0:
            win = c * _L // _W * _W
            gathers.append(pltpu.async_copy(
                ys_hbm.at[idx_v.at[pl.ds(win, _W)], pl.ds(base + win, _W)],
                g_v.at[pl.ds(win, _W)], sem_g))
    outs = []
    for w, cp in enumerate(gathers):
        cp.wait()
        for c in range(w * (_W // _L), (w + 1) * (_W // _L)):
            k = c * _L + lane
            y_v[pl.ds(c * _L, _L)] = plsc.load_gather(g_v, [k, k % _W])
        outs.append(pltpu.async_copy(
            y_v.at[pl.ds(w * _W, _W)],
            out_hbm.at[pl.ds(base + w * _W, _W)], sem_in))
    for cp in outs:
        cp.wait()


def _build(n_grid, n_batch):
    info = plsc.get_sparse_core_info()
    b_per_w = n_batch // info.num_subcores
    mesh = plsc.VectorSubcoreMesh(core_axis_name="c", subcore_axis_name="s",
                                  num_cores=1)
    body = functools.partial(_nn_gather_body, n_grid, n_batch, b_per_w)
    return pl.kernel(
        body,
        mesh=mesh,
        compiler_params=pltpu.CompilerParams(needs_layout_passes=False),
        out_type=jax.ShapeDtypeStruct((n_batch,), jnp.float32),
        scratch_types=[
            pltpu.VMEM((b_per_w,), jnp.float32),
            pltpu.VMEM((b_per_w,), jnp.int32),
            pltpu.VMEM((b_per_w, _W), jnp.float32),
            pltpu.VMEM((b_per_w,), jnp.float32),
            pltpu.SemaphoreType.DMA,
            pltpu.SemaphoreType.DMA,
        ],
    )


def kernel(inputs, ys, ts):
    n_grid, n_batch = ys.shape
    y = _build(n_grid, n_batch)(inputs, ts, ys)
    return y.reshape(-1, 1)


# speedup vs baseline: 1.0005x; 1.0005x over previous
import functools

import jax
import jax.numpy as jnp
from jax import lax
from jax.experimental import pallas as pl
from jax.experimental.pallas import tpu as pltpu
from jax.experimental.pallas import tpu_sc as plsc

_L = 16
_W = 128


def _nn_gather_body(n_grid, n_batch, b_per_w,
                    inp_hbm, ts_hbm, ys_hbm, out_hbm,
                    inp_v, idx_v, g_v, y_v, sem_in, sem_g):
    wid = lax.axis_index("s")
    base = wid * b_per_w
    pltpu.async_copy(inp_hbm.at[pl.ds(base, b_per_w)], inp_v, sem_in).wait()

    j_max = n_grid - 3
    inv_dt = jnp.float32((n_grid - 1) / 100.0)
    dt = jnp.float32(100.0 / (n_grid - 1))
    lane = lax.iota(jnp.int32, _L)

    gathers = []
    for c in range(b_per_w // _L):
        x = inp_v[pl.ds(c * _L, _L)]
        f = (x * inv_dt).astype(jnp.int32) - 1
        m0 = jnp.clip(f, 0, j_max)
        m1 = jnp.clip(f + 1, 0, j_max)
        d0 = jnp.abs(x - (m0 + 1).astype(jnp.float32) * dt)
        d1 = jnp.abs(x - (m1 + 1).astype(jnp.float32) * dt)
        idx_v[pl.ds(c * _L, _L)] = jnp.where(d1 < d0, m1, m0) + 1
        if (c + 1) % (_W // _L) == 0:
            win = c * _L // _W * _W
            gathers.append(pltpu.async_copy(
                ys_hbm.at[idx_v.at[pl.ds(win, _W)], pl.ds(base + win, _W)],
                g_v.at[pl.ds(win, _W)], sem_g))
    for w, cp in enumerate(gathers):
        cp.wait()
        for c in range(w * (_W // _L), (w + 1) * (_W // _L)):
            k = c * _L + lane
            y_v[pl.ds(c * _L, _L)] = plsc.load_gather(g_v, [k, k % _W])
    pltpu.sync_copy(y_v, out_hbm.at[pl.ds(base, b_per_w)])


def _build(n_grid, n_batch):
    info = plsc.get_sparse_core_info()
    b_per_w = n_batch // info.num_subcores
    mesh = plsc.VectorSubcoreMesh(core_axis_name="c", subcore_axis_name="s",
                                  num_cores=1)
    body = functools.partial(_nn_gather_body, n_grid, n_batch, b_per_w)
    return pl.kernel(
        body,
        mesh=mesh,
        compiler_params=pltpu.CompilerParams(needs_layout_passes=False),
        out_type=jax.ShapeDtypeStruct((n_batch,), jnp.float32),
        scratch_types=[
            pltpu.VMEM((b_per_w,), jnp.float32),
            pltpu.VMEM((b_per_w,), jnp.int32),
            pltpu.VMEM((b_per_w, _W), jnp.float32),
            pltpu.VMEM((b_per_w,), jnp.float32),
            pltpu.SemaphoreType.DMA,
            pltpu.SemaphoreType.DMA,
        ],
    )


def kernel(inputs, ys, ts):
    n_grid, n_batch = ys.shape
    y = _build(n_grid, n_batch)(inputs, ts, ys)
    return y.reshape(-1, 1)
